# Initial kernel scaffold; baseline (speedup 1.0000x reference)
import jax
import jax.numpy as jnp
from jax.experimental import pallas as pl


def kernel(zixing_ids, table):
    raise NotImplementedError("write your pallas kernel here")


# speedup vs baseline: 2.2661x; 2.2661x over previous
import functools

import jax
import jax.numpy as jnp
from jax import lax
from jax.experimental import pallas as pl
from jax.experimental.pallas import tpu as pltpu
from jax.experimental.pallas import tpu_sc as plsc

_B, _S, _L, _D = 1024, 20, 4, 768
_VOCAB = 102
_T = _B * _S
_NC, _NS = 2, 16
_NW = _NC * _NS
_TPW = _T // _NW
_CHUNK = 32
_NCHUNK = _TPW // _CHUNK


def _body(ids_hbm, table_hbm, out_hbm, table_v, ids_v, obuf):
    wid = lax.axis_index("s") * _NC + lax.axis_index("c")
    base = wid * _TPW
    pltpu.sync_copy(table_hbm, table_v)
    pltpu.sync_copy(ids_hbm.at[pl.ds(base * _L, _TPW * _L)], ids_v)
    zero = jnp.zeros((16,), jnp.float32)
    for j in range(_D // 16):
        table_v[0, pl.ds(j * 16, 16)] = zero

    def chunk_body(c, carry):
        def grp_body(g, carry2):
            iv = ids_v[pl.ds((c * _CHUNK + g * 4) * _L, 16)]
            for tt in range(4):
                i0 = iv[4 * tt + 0]
                i1 = iv[4 * tt + 1]
                i2 = iv[4 * tt + 2]
                i3 = iv[4 * tt + 3]
                t = g * 4 + tt
                for j in range(_D // 16):
                    ds = pl.ds(j * 16, 16)
                    v = jnp.maximum(
                        jnp.maximum(table_v[i0, ds], table_v[i1, ds]),
                        jnp.maximum(table_v[i2, ds], table_v[i3, ds]))
                    obuf[t, ds] = v
            return carry2

        lax.fori_loop(0, _CHUNK // 4, grp_body, 0)
        pltpu.sync_copy(obuf, out_hbm.at[pl.ds(base + c * _CHUNK, _CHUNK)])
        return carry

    lax.fori_loop(0, _NCHUNK, chunk_body, 0)


@jax.jit
def _glyph(ids_flat, table):
    mesh = plsc.VectorSubcoreMesh(core_axis_name="c", subcore_axis_name="s")
    f = pl.kernel(
        _body,
        out_type=jax.ShapeDtypeStruct((_T, _D), jnp.float32),
        mesh=mesh,
        scratch_types=[
            pltpu.VMEM((_VOCAB, _D), jnp.float32),
            pltpu.VMEM((_TPW * _L,), jnp.int32),
            pltpu.VMEM((_CHUNK, _D), jnp.float32),
        ],
    )
    return f(ids_flat, table)


def kernel(zixing_ids, table):
    ids_flat = zixing_ids.reshape(_T * _L)
    out = _glyph(ids_flat, table)
    return out.reshape(_B, _S, _D)
